# initial kernel scaffold (unmeasured)
import jax
import jax.numpy as jnp
from jax import lax
from jax.experimental import pallas as pl
from jax.experimental.pallas import tpu as pltpu


def kernel(x, W):
    t, d = x.shape
    _, v_loc = W.shape
    v_tot = 2 * v_loc

    def body(x_ref, w_ref, out_ref, send_sem, recv_sem):
        my_x = lax.axis_index("x")
        my_y = lax.axis_index("y")
        my_z = lax.axis_index("z")
        nbr = (my_x, my_y, 1 - my_z)

        barrier = pltpu.get_barrier_semaphore()
        pl.semaphore_signal(
            barrier, inc=1, device_id=nbr,
            device_id_type=pl.DeviceIdType.MESH,
        )
        pl.semaphore_wait(barrier, 1)

        local = lax.dot_general(
            x_ref[:, :], w_ref[:, :],
            dimension_numbers=(((1,), (0,)), ((), ())),
            preferred_element_type=jnp.float32,
        )

        def exchange(lo, hi):
            out_ref[:, lo:hi] = local
            rdma = pltpu.make_async_remote_copy(
                src_ref=out_ref.at[:, pl.ds(lo, v_loc)],
                dst_ref=out_ref.at[:, pl.ds(lo, v_loc)],
                send_sem=send_sem,
                recv_sem=recv_sem,
                device_id=nbr,
                device_id_type=pl.DeviceIdType.MESH,
            )
            rdma.start()
            rdma.wait()

        @pl.when(my_z == 0)
        def _():
            exchange(0, v_loc)

        @pl.when(my_z == 1)
        def _():
            exchange(v_loc, v_tot)

        rows = 128
        for i in range(t // rows):
            l = out_ref[i * rows:(i + 1) * rows, :]
            m = jnp.max(l, axis=1, keepdims=True)
            e = jnp.exp(l - m)
            s = jnp.sum(e, axis=1, keepdims=True)
            out_ref[i * rows:(i + 1) * rows, :] = e / s

    return pl.pallas_call(
        body,
        out_shape=jax.ShapeDtypeStruct((t, v_tot), jnp.float32),
        in_specs=[
            pl.BlockSpec(memory_space=pltpu.VMEM),
            pl.BlockSpec(memory_space=pltpu.VMEM),
        ],
        out_specs=pl.BlockSpec(memory_space=pltpu.VMEM),
        scratch_shapes=[
            pltpu.SemaphoreType.DMA,
            pltpu.SemaphoreType.DMA,
        ],
        compiler_params=pltpu.CompilerParams(collective_id=0),
    )(x, W)


# baseline (device time: 254530 ns/iter reference)
import jax
import jax.numpy as jnp
from jax import lax
from jax.experimental import pallas as pl
from jax.experimental.pallas import tpu as pltpu

V_CHUNK = 1024
ROWS = 64


def kernel(x, W):
    t, d = x.shape
    _, v_loc = W.shape
    v_tot = 2 * v_loc
    n_chunk = v_loc // V_CHUNK
    n_rowb = t // ROWS

    def body(x_ref, w_hbm, out_hbm, w_buf, logits, row_buf,
             w_sems, row_sem, out_sems, send_sems, recv_sems):
        my_x = lax.axis_index("x")
        my_y = lax.axis_index("y")
        my_z = lax.axis_index("z")
        nbr = (my_x, my_y, 1 - my_z)
        my_lo = my_z * v_loc

        barrier = pltpu.get_barrier_semaphore()
        pl.semaphore_signal(
            barrier, inc=1, device_id=nbr,
            device_id_type=pl.DeviceIdType.MESH,
        )
        pl.semaphore_wait(barrier, 1)

        pend = []
        for c in range(n_chunk):
            slot = c % 2
            ld = pltpu.make_async_copy(
                w_hbm.at[:, pl.ds(c * V_CHUNK, V_CHUNK)],
                w_buf.at[slot],
                w_sems.at[slot],
            )
            ld.start()
            ld.wait()
            logits[:, c * V_CHUNK:(c + 1) * V_CHUNK] = lax.dot_general(
                x_ref[:, :], w_buf[slot],
                dimension_numbers=(((1,), (0,)), ((), ())),
                preferred_element_type=jnp.float32,
            )
            src = logits.at[:, pl.ds(c * V_CHUNK, V_CHUNK)]
            dst = out_hbm.at[:, pl.ds(my_lo + c * V_CHUNK, V_CHUNK)]
            rdma = pltpu.make_async_remote_copy(
                src_ref=src, dst_ref=dst,
                send_sem=send_sems.at[c], recv_sem=recv_sems.at[c],
                device_id=nbr, device_id_type=pl.DeviceIdType.MESH,
            )
            rdma.start()
            loc = pltpu.make_async_copy(src, dst, out_sems.at[c])
            loc.start()
            pend.append((rdma, loc))

        for rdma, loc in pend:
            rdma.wait()
            loc.wait()

        for r in range(n_rowb):
            rows = pl.ds(r * ROWS, ROWS)
            ld = pltpu.make_async_copy(out_hbm.at[rows, :], row_buf, row_sem)
            ld.start()
            ld.wait()
            l = row_buf[:, :]
            m = jnp.max(l, axis=1, keepdims=True)
            e = jnp.exp(l - m)
            s = jnp.sum(e, axis=1, keepdims=True)
            row_buf[:, :] = e / s
            st = pltpu.make_async_copy(row_buf, out_hbm.at[rows, :], row_sem)
            st.start()
            st.wait()

    return pl.pallas_call(
        body,
        out_shape=jax.ShapeDtypeStruct((t, v_tot), jnp.float32),
        in_specs=[
            pl.BlockSpec(memory_space=pltpu.VMEM),
            pl.BlockSpec(memory_space=pl.ANY),
        ],
        out_specs=pl.BlockSpec(memory_space=pl.ANY),
        scratch_shapes=[
            pltpu.VMEM((2, d, V_CHUNK), jnp.float32),
            pltpu.VMEM((t, v_loc), jnp.float32),
            pltpu.VMEM((ROWS, v_tot), jnp.float32),
            pltpu.SemaphoreType.DMA((2,)),
            pltpu.SemaphoreType.DMA,
            pltpu.SemaphoreType.DMA((n_chunk,)),
            pltpu.SemaphoreType.DMA((n_chunk,)),
            pltpu.SemaphoreType.DMA((n_chunk,)),
        ],
        compiler_params=pltpu.CompilerParams(collective_id=0),
    )(x, W)


# device time: 239322 ns/iter; 1.0635x vs baseline; 1.0635x over previous
import jax
import jax.numpy as jnp
from jax import lax
from jax.experimental import pallas as pl
from jax.experimental.pallas import tpu as pltpu

V_CHUNK = 512
ROWS = 32


def kernel(x, W):
    t, d = x.shape
    _, v_loc = W.shape
    v_tot = 2 * v_loc
    nc = v_loc // V_CHUNK
    nr = t // ROWS

    def body(x_ref, w_hbm, out_hbm, w_buf, logits, cstage, in_stage, res_buf,
             w_sems, cs_sems, is_sems, rs_sems, send_sems, recv_sems):
        my_x = lax.axis_index("x")
        my_y = lax.axis_index("y")
        my_z = lax.axis_index("z")
        nbr = (my_x, my_y, 1 - my_z)
        my_lo = my_z * v_loc
        nbr_lo = (1 - my_z) * v_loc

        barrier = pltpu.get_barrier_semaphore()
        pl.semaphore_signal(
            barrier, inc=1, device_id=nbr,
            device_id_type=pl.DeviceIdType.MESH,
        )
        pl.semaphore_wait(barrier, 1)

        def w_load(c):
            cp = pltpu.make_async_copy(
                w_hbm.at[:, pl.ds(c * V_CHUNK, V_CHUNK)],
                w_buf.at[c % 2],
                w_sems.at[c % 2],
            )
            cp.start()
            return cp

        rdmas = []
        pending_w = w_load(0)
        m_run = jnp.full((t, 1), -1e30, jnp.float32)
        s_run = jnp.zeros((t, 1), jnp.float32)
        for c in range(nc):
            pending_w.wait()
            if c + 1 < nc:
                pending_w = w_load(c + 1)
            lc = lax.dot_general(
                x_ref[:, :], w_buf[c % 2],
                dimension_numbers=(((1,), (0,)), ((), ())),
                preferred_element_type=jnp.float32,
            )
            logits[:, c * V_CHUNK:(c + 1) * V_CHUNK] = lc
            rdma = pltpu.make_async_remote_copy(
                src_ref=logits.at[:, pl.ds(c * V_CHUNK, V_CHUNK)],
                dst_ref=out_hbm.at[:, pl.ds(my_lo + c * V_CHUNK, V_CHUNK)],
                send_sem=send_sems.at[c], recv_sem=recv_sems.at[c],
                device_id=nbr, device_id_type=pl.DeviceIdType.MESH,
            )
            rdma.start()
            rdmas.append(rdma)
            m_new = jnp.maximum(m_run, jnp.max(lc, axis=1, keepdims=True))
            s_run = s_run * jnp.exp(m_run - m_new) + jnp.sum(
                jnp.exp(lc - m_new), axis=1, keepdims=True)
            m_run = m_new

        def in_load(c):
            rdmas[c].wait_recv()
            cp = pltpu.make_async_copy(
                out_hbm.at[:, pl.ds(nbr_lo + c * V_CHUNK, V_CHUNK)],
                cstage.at[c % 2],
                cs_sems.at[c % 2],
            )
            cp.start()
            return cp

        pending_c = in_load(0)
        for c in range(nc):
            pending_c.wait()
            if c + 1 < nc:
                pending_c = in_load(c + 1)
            lc = cstage[c % 2]
            m_new = jnp.maximum(m_run, jnp.max(lc, axis=1, keepdims=True))
            s_run = s_run * jnp.exp(m_run - m_new) + jnp.sum(
                jnp.exp(lc - m_new), axis=1, keepdims=True)
            m_run = m_new

        inv_s = 1.0 / s_run

        def row_load(r):
            cp = pltpu.make_async_copy(
                out_hbm.at[pl.ds(r * ROWS, ROWS), pl.ds(nbr_lo, v_loc)],
                in_stage.at[r % 2],
                is_sems.at[r % 2],
            )
            cp.start()
            return cp

        pending_r = row_load(0)
        writebacks = [None, None]
        for r in range(nr):
            pending_r.wait()
            if r + 1 < nr:
                pending_r = row_load(r + 1)
            rows = slice(r * ROWS, (r + 1) * ROWS)
            m_r = m_run[rows]
            i_r = inv_s[rows]
            mine = jnp.exp(logits[rows, :] - m_r) * i_r
            other = jnp.exp(in_stage[r % 2] - m_r) * i_r
            if writebacks[r % 2] is not None:
                writebacks[r % 2].wait()

            @pl.when(my_z == 0)
            def _():
                res_buf[r % 2, :, 0:v_loc] = mine
                res_buf[r % 2, :, v_loc:v_tot] = other

            @pl.when(my_z == 1)
            def _():
                res_buf[r % 2, :, 0:v_loc] = other
                res_buf[r % 2, :, v_loc:v_tot] = mine

            wb = pltpu.make_async_copy(
                res_buf.at[r % 2],
                out_hbm.at[pl.ds(r * ROWS, ROWS), :],
                rs_sems.at[r % 2],
            )
            wb.start()
            writebacks[r % 2] = wb

        for wb in writebacks:
            wb.wait()
        for rdma in rdmas:
            rdma.wait_send()

    return pl.pallas_call(
        body,
        out_shape=jax.ShapeDtypeStruct((t, v_tot), jnp.float32),
        in_specs=[
            pl.BlockSpec(memory_space=pltpu.VMEM),
            pl.BlockSpec(memory_space=pl.ANY),
        ],
        out_specs=pl.BlockSpec(memory_space=pl.ANY),
        scratch_shapes=[
            pltpu.VMEM((2, d, V_CHUNK), jnp.float32),
            pltpu.VMEM((t, v_loc), jnp.float32),
            pltpu.VMEM((2, t, V_CHUNK), jnp.float32),
            pltpu.VMEM((2, ROWS, v_loc), jnp.float32),
            pltpu.VMEM((2, ROWS, v_tot), jnp.float32),
            pltpu.SemaphoreType.DMA((2,)),
            pltpu.SemaphoreType.DMA((2,)),
            pltpu.SemaphoreType.DMA((2,)),
            pltpu.SemaphoreType.DMA((2,)),
            pltpu.SemaphoreType.DMA((nc,)),
            pltpu.SemaphoreType.DMA((nc,)),
        ],
        compiler_params=pltpu.CompilerParams(collective_id=0),
    )(x, W)


# device time: 218570 ns/iter; 1.1645x vs baseline; 1.0949x over previous
import jax
import jax.numpy as jnp
from jax import lax
from jax.experimental import pallas as pl
from jax.experimental.pallas import tpu as pltpu

V_CHUNK = 512
F_COLS = 1024
HOLD = 2
BAND = 64


def kernel(x, W):
    t, d = x.shape
    _, v_loc = W.shape
    v_tot = 2 * v_loc
    ncc = v_loc // V_CHUNK
    nf = v_loc // F_COLS
    nb = t // BAND

    def body(x_ref, w_hbm, out_hbm, w_buf, e_my, s_buf, s_in, cstage,
             res_buf, w_sems, cs_sems, rs_sems, st_sems, send_sems,
             recv_sems, s_send, s_recv):
        my_x = lax.axis_index("x")
        my_y = lax.axis_index("y")
        my_z = lax.axis_index("z")
        nbr = (my_x, my_y, 1 - my_z)
        my_lo = my_z * v_loc
        nbr_lo = (1 - my_z) * v_loc

        barrier = pltpu.get_barrier_semaphore()
        pl.semaphore_signal(
            barrier, inc=1, device_id=nbr,
            device_id_type=pl.DeviceIdType.MESH,
        )
        pl.semaphore_wait(barrier, 1)

        def w_load(c):
            cp = pltpu.make_async_copy(
                w_hbm.at[:, pl.ds(c * V_CHUNK, V_CHUNK)],
                w_buf.at[c % 2], w_sems.at[c % 2])
            cp.start()
            return cp

        flows = []
        pending_w = w_load(0)
        s_run = jnp.zeros((t, 1), jnp.float32)
        for c in range(ncc):
            pending_w.wait()
            if c + 1 < ncc:
                pending_w = w_load(c + 1)
            lc = lax.dot_general(
                x_ref[:, :], w_buf[c % 2],
                dimension_numbers=(((1,), (0,)), ((), ())),
                preferred_element_type=jnp.float32)
            e = jnp.exp(lc)
            e_my[:, c * V_CHUNK:(c + 1) * V_CHUNK] = e
            s_run = s_run + jnp.sum(e, axis=1, keepdims=True)
            if (c + 1) * V_CHUNK % F_COLS == 0:
                f = (c + 1) * V_CHUNK // F_COLS - 1
                rdma = pltpu.make_async_remote_copy(
                    src_ref=e_my.at[:, pl.ds(f * F_COLS, F_COLS)],
                    dst_ref=out_hbm.at[:, pl.ds(my_lo + f * F_COLS,
                                                F_COLS)],
                    send_sem=send_sems.at[f], recv_sem=recv_sems.at[f],
                    device_id=nbr, device_id_type=pl.DeviceIdType.MESH)
                flows.append(rdma)
                if f < HOLD:
                    rdma.start()

        s_buf[:, :] = s_run
        s_rdma = pltpu.make_async_remote_copy(
            src_ref=s_buf, dst_ref=s_in,
            send_sem=s_send, recv_sem=s_recv,
            device_id=nbr, device_id_type=pl.DeviceIdType.MESH)
        s_rdma.start()
        for f in range(HOLD, nf):
            flows[f].start()

        s_rdma.wait_recv()
        inv = 1.0 / (s_run + s_in[:, :])

        wb = [None, None]
        for b in range(nb):
            rows = slice(b * BAND, (b + 1) * BAND)
            if wb[b % 2] is not None:
                wb[b % 2].wait()
            res_buf[b % 2, :, :] = e_my[rows, :] * inv[rows]
            cp = pltpu.make_async_copy(
                res_buf.at[b % 2],
                out_hbm.at[pl.ds(b * BAND, BAND), pl.ds(my_lo, v_loc)],
                rs_sems.at[b % 2])
            cp.start()
            wb[b % 2] = cp

        def stage(f):
            flows[f].wait_recv()
            cp = pltpu.make_async_copy(
                out_hbm.at[:, pl.ds(nbr_lo + f * F_COLS, F_COLS)],
                cstage.at[f % 2], cs_sems.at[f % 2])
            cp.start()
            return cp

        stp = [None, None]
        pending_c = stage(0)
        for f in range(nf):
            pending_c.wait()
            if f + 1 < nf:
                if stp[(f + 1) % 2] is not None:
                    stp[(f + 1) % 2].wait()
                    stp[(f + 1) % 2] = None
                pending_c = stage(f + 1)
            cstage[f % 2, :, :] = cstage[f % 2] * inv
            cp = pltpu.make_async_copy(
                cstage.at[f % 2],
                out_hbm.at[:, pl.ds(nbr_lo + f * F_COLS, F_COLS)],
                st_sems.at[f % 2])
            cp.start()
            stp[f % 2] = cp

        for cp in wb + stp:
            if cp is not None:
                cp.wait()
        for rdma in flows:
            rdma.wait_send()
        s_rdma.wait_send()

    return pl.pallas_call(
        body,
        out_shape=jax.ShapeDtypeStruct((t, v_tot), jnp.float32),
        in_specs=[
            pl.BlockSpec(memory_space=pltpu.VMEM),
            pl.BlockSpec(memory_space=pl.ANY),
        ],
        out_specs=pl.BlockSpec(memory_space=pl.ANY),
        scratch_shapes=[
            pltpu.VMEM((2, d, V_CHUNK), jnp.float32),
            pltpu.VMEM((t, v_loc), jnp.float32),
            pltpu.VMEM((t, 1), jnp.float32),
            pltpu.VMEM((t, 1), jnp.float32),
            pltpu.VMEM((2, t, F_COLS), jnp.float32),
            pltpu.VMEM((2, BAND, v_loc), jnp.float32),
            pltpu.SemaphoreType.DMA((2,)),
            pltpu.SemaphoreType.DMA((2,)),
            pltpu.SemaphoreType.DMA((2,)),
            pltpu.SemaphoreType.DMA((2,)),
            pltpu.SemaphoreType.DMA((nf,)),
            pltpu.SemaphoreType.DMA((nf,)),
            pltpu.SemaphoreType.DMA,
            pltpu.SemaphoreType.DMA,
        ],
        compiler_params=pltpu.CompilerParams(collective_id=0),
    )(x, W)


# device time: 139332 ns/iter; 1.8268x vs baseline; 1.5687x over previous
import jax
import jax.numpy as jnp
from jax import lax
from jax.experimental import pallas as pl
from jax.experimental.pallas import tpu as pltpu

V_CHUNK = 512
BAND = 64


def kernel(x, W):
    t, d = x.shape
    _, v_loc = W.shape
    v_tot = 2 * v_loc
    ncc = v_loc // V_CHUNK
    q_cols = v_loc // 4
    nb = t // BAND

    def body(x_ref, w_hbm, out_hbm, w_buf, e_my, s_buf, s_in, cstage,
             res_buf, w_sems, cs_sems, rs_sems, st_sems,
             z_send_sems, z_recv_sems, x_send_sems, x_recv_sems,
             y_send_sems, y_recv_sems, s_send, s_recv):
        my_x = lax.axis_index("x")
        my_y = lax.axis_index("y")
        my_z = lax.axis_index("z")
        z_nbr = (my_x, my_y, 1 - my_z)
        x_nbr = (1 - my_x, my_y, my_z)
        y_nbr = (my_x, 1 - my_y, my_z)
        my_lo = my_z * v_loc
        other_lo = (1 - my_z) * v_loc

        qid_me = 2 * my_x + my_y
        qid_d = 2 * (1 - my_x) + (1 - my_y)
        qid_x = 2 * (1 - my_x) + my_y
        qid_y = 2 * my_x + (1 - my_y)

        barrier = pltpu.get_barrier_semaphore()
        for nbr in (z_nbr, x_nbr, y_nbr):
            pl.semaphore_signal(
                barrier, inc=1, device_id=nbr,
                device_id_type=pl.DeviceIdType.MESH)
        pl.semaphore_wait(barrier, 3)

        chunk_ids = []
        for q in (qid_me, qid_d, qid_x, qid_y):
            for k in range(4):
                chunk_ids.append(q * 4 + k)

        def w_load(i):
            ci = chunk_ids[i]
            cp = pltpu.make_async_copy(
                w_hbm.at[:, pl.ds(ci * V_CHUNK, V_CHUNK)],
                w_buf.at[i % 2], w_sems.at[i % 2])
            cp.start()
            return cp

        def z_send(j, ci):
            rdma = pltpu.make_async_remote_copy(
                src_ref=e_my.at[:, pl.ds(ci * V_CHUNK, V_CHUNK)],
                dst_ref=out_hbm.at[:, pl.ds(my_lo + ci * V_CHUNK,
                                            V_CHUNK)],
                send_sem=z_send_sems.at[j], recv_sem=z_recv_sems.at[j],
                device_id=z_nbr, device_id_type=pl.DeviceIdType.MESH)
            rdma.start()
            return rdma

        z_rdmas = []
        pending_w = w_load(0)
        s_run = jnp.zeros((t, 1), jnp.float32)
        for i in range(ncc):
            ci = chunk_ids[i]
            pending_w.wait()
            if i + 1 < ncc:
                pending_w = w_load(i + 1)
            lc = lax.dot_general(
                x_ref[:, :], w_buf[i % 2],
                dimension_numbers=(((1,), (0,)), ((), ())),
                preferred_element_type=jnp.float32)
            e = jnp.exp(lc)
            e_my[:, pl.ds(ci * V_CHUNK, V_CHUNK)] = e
            s_run = s_run + jnp.sum(e, axis=1, keepdims=True)
            if i < 4:
                z_rdmas.append(z_send(i, ci))

        s_buf[:, :] = s_run
        s_rdma = pltpu.make_async_remote_copy(
            src_ref=s_buf, dst_ref=s_in,
            send_sem=s_send, recv_sem=s_recv,
            device_id=z_nbr, device_id_type=pl.DeviceIdType.MESH)
        s_rdma.start()

        for j in range(4):
            z_rdmas.append(z_send(4 + j, chunk_ids[4 + j]))

        def plane_send(sems_pair, j, abs_col, nbr):
            send_sems_, recv_sems_ = sems_pair
            rdma = pltpu.make_async_remote_copy(
                src_ref=out_hbm.at[:, pl.ds(abs_col, V_CHUNK)],
                dst_ref=out_hbm.at[:, pl.ds(abs_col, V_CHUNK)],
                send_sem=send_sems_.at[j], recv_sem=recv_sems_.at[j],
                device_id=nbr, device_id_type=pl.DeviceIdType.MESH)
            rdma.start()
            return rdma

        x_rdmas = []
        y_rdmas = []
        for j in range(4):
            z_rdmas[j].wait_recv()
            abs_col = other_lo + qid_me * q_cols + j * V_CHUNK
            x_rdmas.append(plane_send((x_send_sems, x_recv_sems), j,
                                      abs_col, x_nbr))
            y_rdmas.append(plane_send((y_send_sems, y_recv_sems), j,
                                      abs_col, y_nbr))

        def plane_recv(sems_pair, j, abs_col, nbr):
            send_sems_, recv_sems_ = sems_pair
            rdma = pltpu.make_async_remote_copy(
                src_ref=out_hbm.at[:, pl.ds(abs_col, V_CHUNK)],
                dst_ref=out_hbm.at[:, pl.ds(abs_col, V_CHUNK)],
                send_sem=send_sems_.at[j], recv_sem=recv_sems_.at[j],
                device_id=nbr, device_id_type=pl.DeviceIdType.MESH)
            rdma.wait_recv()

        for j in range(4):
            plane_recv((x_send_sems, x_recv_sems), j,
                       other_lo + qid_x * q_cols + j * V_CHUNK, x_nbr)
        for j in range(4):
            plane_recv((y_send_sems, y_recv_sems), j,
                       other_lo + qid_y * q_cols + j * V_CHUNK, y_nbr)

        s_rdma.wait_recv()
        inv = 1.0 / (s_run + s_in[:, :])

        stage_slot = [None, None]
        wb_slot = [None, None]
        piece_idx = 0

        def process_piece(abs_col, guard=None):
            nonlocal piece_idx
            sl = piece_idx % 2
            if stage_slot[sl] is not None:
                stage_slot[sl][0].wait()
            if wb_slot[sl] is not None:
                wb_slot[sl][0].wait()
                wb_slot[sl] = None
            cp = pltpu.make_async_copy(
                out_hbm.at[:, pl.ds(abs_col, V_CHUNK)],
                cstage.at[sl], cs_sems.at[sl])
            cp.start()
            cp.wait()
            stage_slot[sl] = None
            cstage[sl, :, :] = cstage[sl] * inv
            if guard is not None:
                guard()
            wb = pltpu.make_async_copy(
                cstage.at[sl],
                out_hbm.at[:, pl.ds(abs_col, V_CHUNK)], st_sems.at[sl])
            wb.start()
            wb_slot[sl] = (wb,)
            piece_idx += 1

        for j in range(4):
            xr, yr = x_rdmas[j], y_rdmas[j]
            process_piece(
                other_lo + qid_me * q_cols + j * V_CHUNK,
                guard=lambda xr=xr, yr=yr: (xr.wait_send(),
                                            yr.wait_send()))
        for j in range(4):
            process_piece(other_lo + qid_x * q_cols + j * V_CHUNK)
        for j in range(4):
            process_piece(other_lo + qid_y * q_cols + j * V_CHUNK)

        band_wb = [None, None]
        for b in range(nb):
            rows = slice(b * BAND, (b + 1) * BAND)
            if band_wb[b % 2] is not None:
                band_wb[b % 2].wait()
            res_buf[b % 2, :, :] = e_my[rows, :] * inv[rows]
            cp = pltpu.make_async_copy(
                res_buf.at[b % 2],
                out_hbm.at[pl.ds(b * BAND, BAND), pl.ds(my_lo, v_loc)],
                rs_sems.at[b % 2])
            cp.start()
            band_wb[b % 2] = cp

        for j in range(4):
            z_rdmas[4 + j].wait_recv()
            process_piece(other_lo + qid_d * q_cols + j * V_CHUNK)

        for cp in band_wb:
            if cp is not None:
                cp.wait()
        for pend in stage_slot + wb_slot:
            if pend is not None:
                pend[0].wait()
        for rdma in z_rdmas:
            rdma.wait_send()
        s_rdma.wait_send()

    return pl.pallas_call(
        body,
        out_shape=jax.ShapeDtypeStruct((t, v_tot), jnp.float32),
        in_specs=[
            pl.BlockSpec(memory_space=pltpu.VMEM),
            pl.BlockSpec(memory_space=pl.ANY),
        ],
        out_specs=pl.BlockSpec(memory_space=pl.ANY),
        scratch_shapes=[
            pltpu.VMEM((2, d, V_CHUNK), jnp.float32),
            pltpu.VMEM((t, v_loc), jnp.float32),
            pltpu.VMEM((t, 1), jnp.float32),
            pltpu.VMEM((t, 1), jnp.float32),
            pltpu.VMEM((2, t, V_CHUNK), jnp.float32),
            pltpu.VMEM((2, BAND, v_loc), jnp.float32),
            pltpu.SemaphoreType.DMA((2,)),
            pltpu.SemaphoreType.DMA((2,)),
            pltpu.SemaphoreType.DMA((2,)),
            pltpu.SemaphoreType.DMA((2,)),
            pltpu.SemaphoreType.DMA((8,)),
            pltpu.SemaphoreType.DMA((8,)),
            pltpu.SemaphoreType.DMA((4,)),
            pltpu.SemaphoreType.DMA((4,)),
            pltpu.SemaphoreType.DMA((4,)),
            pltpu.SemaphoreType.DMA((4,)),
            pltpu.SemaphoreType.DMA,
            pltpu.SemaphoreType.DMA,
        ],
        compiler_params=pltpu.CompilerParams(collective_id=0),
    )(x, W)
